# Initial kernel scaffold; baseline (speedup 1.0000x reference)
import jax
import jax.numpy as jnp
from jax.experimental import pallas as pl


def kernel(x, edge_index, batch, shape_emb, color_emb, pos_emb, W1l, W1r, b1, W2l, W2r, b2, Wc, bc):
    raise NotImplementedError("write your pallas kernel here")


# speedup vs baseline: 6.7306x; 6.7306x over previous
import jax
import jax.numpy as jnp
from jax import lax
from jax.experimental import pallas as pl
from jax.experimental.pallas import tpu as pltpu
from jax.experimental.pallas import tpu_sc as plsc

N = 50000
E = 800000
G = 1024
MAX_POS = 20
NUM_CLASS = 10

NTILE = 16
CHUNK = 128
BLK = 512
NPAD = 50176
NBLK = NPAD // BLK
EPAD = 802816
PEPAD = 53248
GPAD = 1152
DUMMY_NODE = N
DUMMY_GRAPH = G


def _sc_scatter_pass(epad, rpad, ib):
    ept = epad // NTILE
    nck = ept // CHUNK
    nblk = nck // ib
    rpt = rpad // NTILE
    mesh = plsc.VectorSubcoreMesh(core_axis_name="c", subcore_axis_name="s")

    def body(src2, dst2, flo, fhi, z2,
             acc_lo, acc_hi,
             acc_sh, src_v, dst_v, buf0, buf1, sem0, sem1):
        c = lax.axis_index("c")
        s = lax.axis_index("s")
        base = s * nck
        pltpu.sync_copy(z2.at[pl.ds(s * rpt, rpt)], acc_sh.at[pl.ds(s * rpt, rpt)])
        plsc.subcore_barrier()

        def run(feat, acc_out):
            bufs = (buf0, buf1)
            sems = (sem0, sem1)

            @pl.loop(0, nblk)
            def _(t):
                blk = base + t * ib
                pltpu.sync_copy(src2.at[pl.ds(blk, ib)], src_v)
                pltpu.sync_copy(dst2.at[pl.ds(blk, ib)], dst_v)
                for b in range(min(2, ib)):
                    pltpu.async_copy(feat.at[src_v.at[b]], bufs[b], sems[b])
                for b in range(ib):
                    pltpu.make_async_copy(feat.at[src_v.at[b]], bufs[b % 2],
                                          sems[b % 2]).wait()
                    pltpu.sync_copy(bufs[b % 2], acc_sh.at[dst_v.at[b]], add=True)
                    if b + 2 < ib:
                        pltpu.async_copy(feat.at[src_v.at[b + 2]], bufs[b % 2],
                                         sems[b % 2])

            plsc.subcore_barrier()
            pltpu.sync_copy(acc_sh.at[pl.ds(s * rpt, rpt)],
                            acc_out.at[pl.ds(s * rpt, rpt)])

        @pl.when(c == 0)
        def _():
            run(flo, acc_lo)

        @pl.when(c == 1)
        def _():
            run(fhi, acc_hi)

    f32 = jnp.float32
    return pl.kernel(
        body,
        out_type=(jax.ShapeDtypeStruct((rpad, 32), f32),
                  jax.ShapeDtypeStruct((rpad, 32), f32)),
        mesh=mesh,
        compiler_params=pltpu.CompilerParams(use_tc_tiling_on_sc=False),
        scratch_types=(
            pltpu.VMEM_SHARED((rpad, 32), f32),
            pltpu.VMEM((ib, CHUNK), jnp.int32),
            pltpu.VMEM((ib, CHUNK), jnp.int32),
            pltpu.VMEM((CHUNK, 32), f32),
            pltpu.VMEM((CHUNK, 32), f32),
            pltpu.SemaphoreType.DMA,
            pltpu.SemaphoreType.DMA,
        ),
    )


def _sc_hist_pass():
    nck_e = EPAD // NTILE // CHUNK
    nck_p = PEPAD // NTILE // CHUNK
    rpt_n = NPAD // NTILE
    rpt_g = GPAD // NTILE
    mesh = plsc.VectorSubcoreMesh(core_axis_name="c", subcore_axis_name="s")

    def body(dst2, pdst2, zn16, zg16, ones_h,
             cnt_out, gcnt_out,
             cnt_sh, gcnt_sh, dst_v, ones_v, sem):
        c = lax.axis_index("c")
        s = lax.axis_index("s")
        pltpu.sync_copy(ones_h, ones_v)

        def run(idx2, sh, out, nck, ib, rpt):
            base = s * nck
            nblk = nck // ib

            @pl.loop(0, nblk)
            def _(t):
                pltpu.sync_copy(idx2.at[pl.ds(base + t * ib, ib)], dst_v.at[pl.ds(0, ib)])
                for b in range(ib):
                    pltpu.async_copy(ones_v, sh.at[dst_v.at[b]], sem, add=True)
                for b in range(ib):
                    pltpu.make_async_copy(ones_v, sh.at[dst_v.at[b]], sem).wait()

            plsc.subcore_barrier()
            pltpu.sync_copy(sh.at[pl.ds(s * rpt, rpt)], out.at[pl.ds(s * rpt, rpt)])

        @pl.when(c == 0)
        def _():
            pltpu.sync_copy(zn16.at[pl.ds(s * rpt_n, rpt_n)],
                            cnt_sh.at[pl.ds(s * rpt_n, rpt_n)])
            plsc.subcore_barrier()
            run(dst2, cnt_sh, cnt_out, nck_e, 8, rpt_n)

        @pl.when(c == 1)
        def _():
            pltpu.sync_copy(zg16.at[pl.ds(s * rpt_g, rpt_g)],
                            gcnt_sh.at[pl.ds(s * rpt_g, rpt_g)])
            plsc.subcore_barrier()
            run(pdst2, gcnt_sh, gcnt_out, nck_p, 2, rpt_g)

    f32 = jnp.float32
    return pl.kernel(
        body,
        out_type=(jax.ShapeDtypeStruct((NPAD, 16), f32),
                  jax.ShapeDtypeStruct((GPAD, 16), f32)),
        mesh=mesh,
        compiler_params=pltpu.CompilerParams(use_tc_tiling_on_sc=False),
        scratch_types=(
            pltpu.VMEM_SHARED((NPAD, 16), f32),
            pltpu.VMEM_SHARED((GPAD, 16), f32),
            pltpu.VMEM((8, CHUNK), jnp.int32),
            pltpu.VMEM((CHUNK, 16), f32),
            pltpu.SemaphoreType.DMA,
        ),
    )


def _embed_body(x0, x1, x2, bmat, lo_ref, hi_ref):
    v0 = x0[0, 0, :]
    v1 = x1[0, 0, :]
    v2 = jnp.clip(x2[0, 0, :], 0, MAX_POS - 1)
    i8 = lax.broadcasted_iota(jnp.int32, (BLK, 8), 1)
    i20 = lax.broadcasted_iota(jnp.int32, (BLK, 20), 1)
    oh = jnp.concatenate(
        [(v0[:, None] == i8).astype(jnp.float32),
         (v1[:, None] == i8).astype(jnp.float32),
         (v2[:, None] == i20).astype(jnp.float32),
         jnp.zeros((BLK, 4), jnp.float32)], axis=1)
    feat = jnp.dot(oh, bmat[...], preferred_element_type=jnp.float32)
    lo_ref[...] = feat[:, :32]
    hi_ref[...] = feat[:, 32:]


def _sage_body(alo, ahi, flo, fhi, cnt3, wl, wr, b, olo_ref, ohi_ref):
    acc = jnp.concatenate([alo[...], ahi[...]], axis=1)
    f = jnp.concatenate([flo[...], fhi[...]], axis=1)
    inv = 1.0 / jnp.maximum(cnt3[0, 0, :], 1.0)
    h = (jnp.dot(acc, wl[...], preferred_element_type=jnp.float32) * inv[:, None]
         + jnp.dot(f, wr[...], preferred_element_type=jnp.float32) + b[...])
    h = jnp.maximum(h, 0.0)
    olo_ref[...] = h[:, :32]
    ohi_ref[...] = h[:, 32:]


def _head_body(alo, ahi, g3, wc, bc, out_ref):
    acc = jnp.concatenate([alo[...], ahi[...]], axis=1)
    g = jnp.reshape(g3[0], (GPAD,))
    inv = 1.0 / jnp.maximum(g, 1.0)
    hg = acc * inv[:, None]
    o = jnp.dot(hg, wc[...], preferred_element_type=jnp.float32) + bc[...]
    out_ref[...] = o[:G, :]


def _row_spec():
    return pl.BlockSpec((BLK, 32), lambda i: (i, 0))


def _full_spec(shape):
    nd = len(shape)
    return pl.BlockSpec(shape, lambda i: (0,) * nd)


def kernel(x, edge_index, batch, shape_emb, color_emb, pos_emb,
           W1l, W1r, b1, W2l, W2r, b2, Wc, bc):
    f32 = jnp.float32
    i32 = jnp.int32

    xp = jnp.concatenate([x.astype(i32), jnp.zeros((NPAD - N, 3), i32)], axis=0)
    x0 = xp[:, 0].reshape(NBLK, 1, BLK)
    x1 = xp[:, 1].reshape(NBLK, 1, BLK)
    x2 = xp[:, 2].reshape(NBLK, 1, BLK)

    bmat = jnp.zeros((40, 64), f32)
    bmat = bmat.at[0:8, 0:16].set(shape_emb)
    bmat = bmat.at[8:16, 16:32].set(color_emb)
    bmat = bmat.at[16:36, 32:48].set(pos_emb)

    w1l = jnp.zeros((64, 64), f32).at[:48].set(W1l)
    w1r = jnp.zeros((64, 64), f32).at[:48].set(W1r)
    b1r = b1.reshape(1, 64)
    b2r = b2.reshape(1, 64)
    wc = jnp.zeros((64, 16), f32).at[:, :NUM_CLASS].set(Wc)
    bcr = jnp.zeros((1, 16), f32).at[0, :NUM_CLASS].set(bc)

    src = edge_index[0].astype(i32)
    dst = edge_index[1].astype(i32)
    src2 = jnp.concatenate([src, jnp.zeros((EPAD - E,), i32)]).reshape(EPAD // CHUNK, CHUNK)
    dst2 = jnp.concatenate([dst, jnp.full((EPAD - E,), DUMMY_NODE, i32)]).reshape(EPAD // CHUNK, CHUNK)

    psrc = jnp.concatenate([jnp.arange(N, dtype=i32),
                            jnp.zeros((PEPAD - N,), i32)]).reshape(PEPAD // CHUNK, CHUNK)
    pdst = jnp.concatenate([batch.astype(i32),
                            jnp.full((PEPAD - N,), DUMMY_GRAPH, i32)]).reshape(PEPAD // CHUNK, CHUNK)

    ones16 = jnp.ones((CHUNK, 16), f32)
    zn2 = jnp.zeros((NPAD, 32), f32)
    zn16 = jnp.zeros((NPAD, 16), f32)
    zg2 = jnp.zeros((GPAD, 32), f32)
    zg16 = jnp.zeros((GPAD, 16), f32)

    embed = pl.pallas_call(
        _embed_body,
        grid=(NBLK,),
        in_specs=[pl.BlockSpec((1, 1, BLK), lambda i: (i, 0, 0))] * 3
        + [_full_spec((40, 64))],
        out_specs=[_row_spec(), _row_spec()],
        out_shape=[jax.ShapeDtypeStruct((NPAD, 32), f32)] * 2,
    )
    f0lo, f0hi = embed(x0, x1, x2, bmat)

    hist = _sc_hist_pass()
    cnt16, gcnt16 = hist(dst2, pdst, zn16, zg16, ones16)

    edge_pass = _sc_scatter_pass(EPAD, NPAD, 8)
    a1lo, a1hi = edge_pass(src2, dst2, f0lo, f0hi, zn2)
    cnt3 = cnt16[:, 0].reshape(NBLK, 1, BLK)

    sage = pl.pallas_call(
        _sage_body,
        grid=(NBLK,),
        in_specs=[_row_spec()] * 4
        + [pl.BlockSpec((1, 1, BLK), lambda i: (i, 0, 0)),
           _full_spec((64, 64)), _full_spec((64, 64)), _full_spec((1, 64))],
        out_specs=[_row_spec(), _row_spec()],
        out_shape=[jax.ShapeDtypeStruct((NPAD, 32), f32)] * 2,
    )
    h1lo, h1hi = sage(a1lo, a1hi, f0lo, f0hi, cnt3, w1l, w1r, b1r)

    a2lo, a2hi = edge_pass(src2, dst2, h1lo, h1hi, zn2)

    h2lo, h2hi = sage(a2lo, a2hi, h1lo, h1hi, cnt3, W2l, W2r, b2r)

    pool_pass = _sc_scatter_pass(PEPAD, GPAD, 2)
    aplo, aphi = pool_pass(psrc, pdst, h2lo, h2hi, zg2)
    g3 = gcnt16[:, 0].reshape(1, GPAD // CHUNK, CHUNK)

    head = pl.pallas_call(
        _head_body,
        grid=(1,),
        in_specs=[_full_spec((GPAD, 32)), _full_spec((GPAD, 32)),
                  _full_spec((1, GPAD // CHUNK, CHUNK)),
                  _full_spec((64, 16)), _full_spec((1, 16))],
        out_specs=_full_spec((G, 16)),
        out_shape=jax.ShapeDtypeStruct((G, 16), f32),
    )
    out = head(aplo, aphi, g3, wc, bcr)
    return out[:, :NUM_CLASS]


# speedup vs baseline: 9.4583x; 1.4053x over previous
import jax
import jax.numpy as jnp
from jax import lax
from jax.experimental import pallas as pl
from jax.experimental.pallas import tpu as pltpu
from jax.experimental.pallas import tpu_sc as plsc

N = 50000
E = 800000
G = 1024
MAX_POS = 20
NUM_CLASS = 10

NTILE = 16
CHUNK = 128
BLK = 512
NPAD = 50176
NBLK = NPAD // BLK
EPAD = 811008
PEPAD = 53248
GPAD = 1152
DUMMY_NODE = N
DUMMY_GRAPH = G


def _sc_scatter_pass(epad, rpad, ib):
    ept = epad // NTILE
    nck = ept // CHUNK
    nblk = nck // ib
    rpt = rpad // NTILE
    mesh = plsc.VectorSubcoreMesh(core_axis_name="c", subcore_axis_name="s")

    def body(src2, dst2, flo, fhi, z2,
             acc_lo, acc_hi,
             acc_sh, src_v, dst_v, bufs, gsems, ssems, isem):
        c = lax.axis_index("c")
        s = lax.axis_index("s")
        base = s * nck
        pltpu.sync_copy(z2.at[pl.ds(s * rpt, rpt)], acc_sh.at[pl.ds(s * rpt, rpt)])
        plsc.subcore_barrier()

        def load_idx(t, slot):
            blk = base + t * ib
            pltpu.async_copy(src2.at[pl.ds(blk, ib)], src_v.at[slot], isem)
            pltpu.async_copy(dst2.at[pl.ds(blk, ib)], dst_v.at[slot], isem)

        def wait_idx(slot):
            pltpu.make_async_copy(src2.at[pl.ds(base, ib)], src_v.at[slot], isem).wait()
            pltpu.make_async_copy(dst2.at[pl.ds(base, ib)], dst_v.at[slot], isem).wait()

        def run(feat, acc_out):
            load_idx(0, 0)
            wait_idx(0)
            if nblk > 1:
                load_idx(1, 1)
            for b in range(ib):
                pltpu.async_copy(feat.at[src_v.at[0].at[b]], bufs[b], gsems[b])

            @pl.loop(0, nblk)
            def _(t):
                slot = t % 2
                nslot = (t + 1) % 2
                for b in range(ib):
                    pltpu.make_async_copy(feat.at[src_v.at[slot].at[b]],
                                          bufs[b], gsems[b]).wait()
                    pltpu.async_copy(bufs[b], acc_sh.at[dst_v.at[slot].at[b]],
                                     ssems[b], add=True)

                @pl.when(t + 1 < nblk)
                def _():
                    wait_idx(nslot)
                    for b in range(ib):
                        pltpu.make_async_copy(bufs[b],
                                              acc_sh.at[dst_v.at[slot].at[b]],
                                              ssems[b]).wait()
                        pltpu.async_copy(feat.at[src_v.at[nslot].at[b]],
                                         bufs[b], gsems[b])

                    @pl.when(t + 2 < nblk)
                    def _():
                        load_idx(t + 2, slot)

                @pl.when(t + 1 >= nblk)
                def _():
                    for b in range(ib):
                        pltpu.make_async_copy(bufs[b],
                                              acc_sh.at[dst_v.at[slot].at[b]],
                                              ssems[b]).wait()

            plsc.subcore_barrier()
            pltpu.sync_copy(acc_sh.at[pl.ds(s * rpt, rpt)],
                            acc_out.at[pl.ds(s * rpt, rpt)])

        @pl.when(c == 0)
        def _():
            run(flo, acc_lo)

        @pl.when(c == 1)
        def _():
            run(fhi, acc_hi)

    f32 = jnp.float32
    return pl.kernel(
        body,
        out_type=(jax.ShapeDtypeStruct((rpad, 32), f32),
                  jax.ShapeDtypeStruct((rpad, 32), f32)),
        mesh=mesh,
        compiler_params=pltpu.CompilerParams(use_tc_tiling_on_sc=False),
        scratch_types=(
            pltpu.VMEM_SHARED((rpad, 32), f32),
            pltpu.VMEM((2, ib, CHUNK), jnp.int32),
            pltpu.VMEM((2, ib, CHUNK), jnp.int32),
            [pltpu.VMEM((CHUNK, 32), f32) for _ in range(ib)],
            [pltpu.SemaphoreType.DMA for _ in range(ib)],
            [pltpu.SemaphoreType.DMA for _ in range(ib)],
            pltpu.SemaphoreType.DMA,
        ),
    )


def _sc_hist_pass():
    nck_e = EPAD // NTILE // CHUNK
    nck_p = PEPAD // NTILE // CHUNK
    rpt_n = NPAD // NTILE
    rpt_g = GPAD // NTILE
    mesh = plsc.VectorSubcoreMesh(core_axis_name="c", subcore_axis_name="s")

    def body(dst2, pdst2, zn16, zg16, ones_h,
             cnt_out, gcnt_out,
             cnt_sh, gcnt_sh, dst_v, ones_v, sem):
        c = lax.axis_index("c")
        s = lax.axis_index("s")
        pltpu.sync_copy(ones_h, ones_v)

        def run(idx2, sh, out, nck, ib, rpt):
            base = s * nck
            nblk = nck // ib

            @pl.loop(0, nblk)
            def _(t):
                pltpu.sync_copy(idx2.at[pl.ds(base + t * ib, ib)], dst_v.at[pl.ds(0, ib)])
                for b in range(ib):
                    pltpu.async_copy(ones_v, sh.at[dst_v.at[b]], sem, add=True)
                for b in range(ib):
                    pltpu.make_async_copy(ones_v, sh.at[dst_v.at[b]], sem).wait()

            plsc.subcore_barrier()
            pltpu.sync_copy(sh.at[pl.ds(s * rpt, rpt)], out.at[pl.ds(s * rpt, rpt)])

        @pl.when(c == 0)
        def _():
            pltpu.sync_copy(zn16.at[pl.ds(s * rpt_n, rpt_n)],
                            cnt_sh.at[pl.ds(s * rpt_n, rpt_n)])
            plsc.subcore_barrier()
            run(dst2, cnt_sh, cnt_out, nck_e, 6, rpt_n)

        @pl.when(c == 1)
        def _():
            pltpu.sync_copy(zg16.at[pl.ds(s * rpt_g, rpt_g)],
                            gcnt_sh.at[pl.ds(s * rpt_g, rpt_g)])
            plsc.subcore_barrier()
            run(pdst2, gcnt_sh, gcnt_out, nck_p, 2, rpt_g)

    f32 = jnp.float32
    return pl.kernel(
        body,
        out_type=(jax.ShapeDtypeStruct((NPAD, 16), f32),
                  jax.ShapeDtypeStruct((GPAD, 16), f32)),
        mesh=mesh,
        compiler_params=pltpu.CompilerParams(use_tc_tiling_on_sc=False),
        scratch_types=(
            pltpu.VMEM_SHARED((NPAD, 16), f32),
            pltpu.VMEM_SHARED((GPAD, 16), f32),
            pltpu.VMEM((8, CHUNK), jnp.int32),
            pltpu.VMEM((CHUNK, 16), f32),
            pltpu.SemaphoreType.DMA,
        ),
    )


def _embed_body(x0, x1, x2, bmat, lo_ref, hi_ref):
    v0 = x0[0, 0, :]
    v1 = x1[0, 0, :]
    v2 = jnp.clip(x2[0, 0, :], 0, MAX_POS - 1)
    i8 = lax.broadcasted_iota(jnp.int32, (BLK, 8), 1)
    i20 = lax.broadcasted_iota(jnp.int32, (BLK, 20), 1)
    oh = jnp.concatenate(
        [(v0[:, None] == i8).astype(jnp.float32),
         (v1[:, None] == i8).astype(jnp.float32),
         (v2[:, None] == i20).astype(jnp.float32),
         jnp.zeros((BLK, 4), jnp.float32)], axis=1)
    feat = jnp.dot(oh, bmat[...], preferred_element_type=jnp.float32)
    lo_ref[...] = feat[:, :32]
    hi_ref[...] = feat[:, 32:]


def _sage_body(alo, ahi, flo, fhi, cnt3, wl, wr, b, olo_ref, ohi_ref):
    acc = jnp.concatenate([alo[...], ahi[...]], axis=1)
    f = jnp.concatenate([flo[...], fhi[...]], axis=1)
    inv = 1.0 / jnp.maximum(cnt3[0, 0, :], 1.0)
    h = (jnp.dot(acc, wl[...], preferred_element_type=jnp.float32) * inv[:, None]
         + jnp.dot(f, wr[...], preferred_element_type=jnp.float32) + b[...])
    h = jnp.maximum(h, 0.0)
    olo_ref[...] = h[:, :32]
    ohi_ref[...] = h[:, 32:]


def _head_body(alo, ahi, g3, wc, bc, out_ref):
    acc = jnp.concatenate([alo[...], ahi[...]], axis=1)
    g = jnp.reshape(g3[0], (GPAD,))
    inv = 1.0 / jnp.maximum(g, 1.0)
    hg = acc * inv[:, None]
    o = jnp.dot(hg, wc[...], preferred_element_type=jnp.float32) + bc[...]
    out_ref[...] = o[:G, :]


def _row_spec():
    return pl.BlockSpec((BLK, 32), lambda i: (i, 0))


def _full_spec(shape):
    nd = len(shape)
    return pl.BlockSpec(shape, lambda i: (0,) * nd)


def kernel(x, edge_index, batch, shape_emb, color_emb, pos_emb,
           W1l, W1r, b1, W2l, W2r, b2, Wc, bc):
    f32 = jnp.float32
    i32 = jnp.int32

    xp = jnp.concatenate([x.astype(i32), jnp.zeros((NPAD - N, 3), i32)], axis=0)
    x0 = xp[:, 0].reshape(NBLK, 1, BLK)
    x1 = xp[:, 1].reshape(NBLK, 1, BLK)
    x2 = xp[:, 2].reshape(NBLK, 1, BLK)

    bmat = jnp.zeros((40, 64), f32)
    bmat = bmat.at[0:8, 0:16].set(shape_emb)
    bmat = bmat.at[8:16, 16:32].set(color_emb)
    bmat = bmat.at[16:36, 32:48].set(pos_emb)

    w1l = jnp.zeros((64, 64), f32).at[:48].set(W1l)
    w1r = jnp.zeros((64, 64), f32).at[:48].set(W1r)
    b1r = b1.reshape(1, 64)
    b2r = b2.reshape(1, 64)
    wc = jnp.zeros((64, 16), f32).at[:, :NUM_CLASS].set(Wc)
    bcr = jnp.zeros((1, 16), f32).at[0, :NUM_CLASS].set(bc)

    src = edge_index[0].astype(i32)
    dst = edge_index[1].astype(i32)
    pad_cycle = jnp.arange(EPAD - E, dtype=i32) % 128
    src2 = jnp.concatenate([src, pad_cycle]).reshape(EPAD // CHUNK, CHUNK)
    dst2 = jnp.concatenate([dst, DUMMY_NODE + pad_cycle]).reshape(EPAD // CHUNK, CHUNK)

    ppad_cycle = jnp.arange(PEPAD - N, dtype=i32) % 128
    psrc = jnp.concatenate([jnp.arange(N, dtype=i32),
                            ppad_cycle]).reshape(PEPAD // CHUNK, CHUNK)
    pdst = jnp.concatenate([batch.astype(i32),
                            DUMMY_GRAPH + ppad_cycle]).reshape(PEPAD // CHUNK, CHUNK)

    ones16 = jnp.ones((CHUNK, 16), f32)
    zn2 = jnp.zeros((NPAD, 32), f32)
    zn16 = jnp.zeros((NPAD, 16), f32)
    zg2 = jnp.zeros((GPAD, 32), f32)
    zg16 = jnp.zeros((GPAD, 16), f32)

    embed = pl.pallas_call(
        _embed_body,
        grid=(NBLK,),
        in_specs=[pl.BlockSpec((1, 1, BLK), lambda i: (i, 0, 0))] * 3
        + [_full_spec((40, 64))],
        out_specs=[_row_spec(), _row_spec()],
        out_shape=[jax.ShapeDtypeStruct((NPAD, 32), f32)] * 2,
    )
    f0lo, f0hi = embed(x0, x1, x2, bmat)

    hist = _sc_hist_pass()
    cnt16, gcnt16 = hist(dst2, pdst, zn16, zg16, ones16)

    edge_pass = _sc_scatter_pass(EPAD, NPAD, 6)
    a1lo, a1hi = edge_pass(src2, dst2, f0lo, f0hi, zn2)
    cnt3 = cnt16[:, 0].reshape(NBLK, 1, BLK)

    sage = pl.pallas_call(
        _sage_body,
        grid=(NBLK,),
        in_specs=[_row_spec()] * 4
        + [pl.BlockSpec((1, 1, BLK), lambda i: (i, 0, 0)),
           _full_spec((64, 64)), _full_spec((64, 64)), _full_spec((1, 64))],
        out_specs=[_row_spec(), _row_spec()],
        out_shape=[jax.ShapeDtypeStruct((NPAD, 32), f32)] * 2,
    )
    h1lo, h1hi = sage(a1lo, a1hi, f0lo, f0hi, cnt3, w1l, w1r, b1r)

    a2lo, a2hi = edge_pass(src2, dst2, h1lo, h1hi, zn2)

    h2lo, h2hi = sage(a2lo, a2hi, h1lo, h1hi, cnt3, W2l, W2r, b2r)

    pool_pass = _sc_scatter_pass(PEPAD, GPAD, 2)
    aplo, aphi = pool_pass(psrc, pdst, h2lo, h2hi, zg2)
    g3 = gcnt16[:, 0].reshape(1, GPAD // CHUNK, CHUNK)

    head = pl.pallas_call(
        _head_body,
        grid=(1,),
        in_specs=[_full_spec((GPAD, 32)), _full_spec((GPAD, 32)),
                  _full_spec((1, GPAD // CHUNK, CHUNK)),
                  _full_spec((64, 16)), _full_spec((1, 16))],
        out_specs=_full_spec((G, 16)),
        out_shape=jax.ShapeDtypeStruct((G, 16), f32),
    )
    out = head(aplo, aphi, g3, wc, bcr)
    return out[:, :NUM_CLASS]


# speedup vs baseline: 14.4840x; 1.5314x over previous
import numpy as np
import jax
import jax.numpy as jnp
from jax import lax
from jax.experimental import pallas as pl
from jax.experimental.pallas import tpu as pltpu
from jax.experimental.pallas import tpu_sc as plsc

N = 50000
E = 800000
G = 1024
MAX_POS = 20
NUM_CLASS = 10

NTILE = 16
CHUNK = 128
BLK = 3584
PBLK = BLK // 4
NPAD = 50176
NBLK = NPAD // BLK
EPAD = 811008
PEPAD = 53248
GPAD = 1152
DUMMY_NODE = N
DUMMY_GRAPH = G

_CMAP = np.array([32 * (l // 128) + (l % 32) for l in range(256)])
_KMAP = np.array([(l % 128) // 32 for l in range(256)])
_KEQ = (_KMAP[:, None] == _KMAP[None, :]).astype(np.float32)

_EKMAP = np.array([l // 40 for l in range(160)])
_EMMAP = np.array([l % 40 for l in range(160)])
_EKEQ = (_EKMAP[:, None] == _KMAP[None, :]).astype(np.float32)


def _pack_w(w):
    return w[jnp.asarray(_CMAP)][:, jnp.asarray(_CMAP)] * _KEQ


def _pack_b(b):
    return b[jnp.asarray(_CMAP)].reshape(1, 256)


def _pack_bmat(bmat):
    return bmat[jnp.asarray(_EMMAP)][:, jnp.asarray(_CMAP)] * _EKEQ


def _sc_scatter_pass(epad, rpad, ib):
    ept = epad // NTILE
    nck = ept // CHUNK
    nblk = nck // ib
    rpt = rpad // NTILE
    mesh = plsc.VectorSubcoreMesh(core_axis_name="c", subcore_axis_name="s")

    def body(src2, dst2, flo, fhi, z2,
             acc_lo, acc_hi,
             acc_sh, src_v, dst_v, bufs, gsems, ssems, isem):
        c = lax.axis_index("c")
        s = lax.axis_index("s")
        base = s * nck
        pltpu.sync_copy(z2.at[pl.ds(0, rpt)], acc_sh.at[pl.ds(s * rpt, rpt)])
        plsc.subcore_barrier()

        def load_idx(t, slot):
            blk = base + t * ib
            pltpu.async_copy(src2.at[pl.ds(blk, ib)], src_v.at[slot], isem)
            pltpu.async_copy(dst2.at[pl.ds(blk, ib)], dst_v.at[slot], isem)

        def wait_idx(slot):
            pltpu.make_async_copy(src2.at[pl.ds(base, ib)], src_v.at[slot], isem).wait()
            pltpu.make_async_copy(dst2.at[pl.ds(base, ib)], dst_v.at[slot], isem).wait()

        def run(feat, acc_out):
            load_idx(0, 0)
            wait_idx(0)
            if nblk > 1:
                load_idx(1, 1)
            for b in range(ib):
                pltpu.async_copy(feat.at[src_v.at[0].at[b]], bufs[b], gsems[b])

            @pl.loop(0, nblk)
            def _(t):
                slot = t % 2
                nslot = (t + 1) % 2
                for b in range(ib):
                    pltpu.make_async_copy(feat.at[src_v.at[slot].at[b]],
                                          bufs[b], gsems[b]).wait()
                    pltpu.async_copy(bufs[b], acc_sh.at[dst_v.at[slot].at[b]],
                                     ssems[b], add=True)

                @pl.when(t + 1 < nblk)
                def _():
                    wait_idx(nslot)
                    for b in range(ib):
                        pltpu.make_async_copy(bufs[b],
                                              acc_sh.at[dst_v.at[slot].at[b]],
                                              ssems[b]).wait()
                        pltpu.async_copy(feat.at[src_v.at[nslot].at[b]],
                                         bufs[b], gsems[b])

                    @pl.when(t + 2 < nblk)
                    def _():
                        load_idx(t + 2, slot)

                @pl.when(t + 1 >= nblk)
                def _():
                    for b in range(ib):
                        pltpu.make_async_copy(bufs[b],
                                              acc_sh.at[dst_v.at[slot].at[b]],
                                              ssems[b]).wait()

            plsc.subcore_barrier()
            pltpu.sync_copy(acc_sh.at[pl.ds(s * rpt, rpt)],
                            acc_out.at[pl.ds(s * rpt, rpt)])

        @pl.when(c == 0)
        def _():
            run(flo, acc_lo)

        @pl.when(c == 1)
        def _():
            run(fhi, acc_hi)

    f32 = jnp.float32
    return pl.kernel(
        body,
        out_type=(jax.ShapeDtypeStruct((rpad, 32), f32),
                  jax.ShapeDtypeStruct((rpad, 32), f32)),
        mesh=mesh,
        compiler_params=pltpu.CompilerParams(use_tc_tiling_on_sc=False),
        scratch_types=(
            pltpu.VMEM_SHARED((rpad, 32), f32),
            pltpu.VMEM((2, ib, CHUNK), jnp.int32),
            pltpu.VMEM((2, ib, CHUNK), jnp.int32),
            [pltpu.VMEM((CHUNK, 32), f32) for _ in range(ib)],
            [pltpu.SemaphoreType.DMA for _ in range(ib)],
            [pltpu.SemaphoreType.DMA for _ in range(ib)],
            pltpu.SemaphoreType.DMA,
        ),
    )


def _sc_hist_pass():
    nck_e = EPAD // NTILE // CHUNK // 2
    nck_p = PEPAD // NTILE // CHUNK
    rpt_n = NPAD // NTILE
    rpt_g = GPAD // NTILE
    mesh = plsc.VectorSubcoreMesh(core_axis_name="c", subcore_axis_name="s")

    def body(dst2, pdst2, z2, ones_h,
             cnt_a, cnt_b, gcnt_out,
             cnt_sh, gcnt_sh, dst_v, ones_v, sem):
        c = lax.axis_index("c")
        s = lax.axis_index("s")
        pltpu.sync_copy(ones_h, ones_v)

        def run(idx2, off, sh, out, nck, ib, rpt):
            base = off + s * nck
            nblk = nck // ib

            @pl.loop(0, nblk)
            def _(t):
                pltpu.sync_copy(idx2.at[pl.ds(base + t * ib, ib)],
                                dst_v.at[pl.ds(0, ib)])
                for b in range(ib):
                    pltpu.async_copy(ones_v, sh.at[dst_v.at[b]], sem, add=True)
                for b in range(ib):
                    pltpu.make_async_copy(ones_v, sh.at[dst_v.at[b]], sem).wait()

            plsc.subcore_barrier()
            pltpu.sync_copy(sh.at[pl.ds(s * rpt, rpt)], out.at[pl.ds(s * rpt, rpt)])

        pltpu.sync_copy(z2.at[pl.ds(0, rpt_n)], cnt_sh.at[pl.ds(s * rpt_n, rpt_n)])

        @pl.when(c == 0)
        def _():
            plsc.subcore_barrier()
            run(dst2, 0, cnt_sh, cnt_a, nck_e, 6, rpt_n)

        @pl.when(c == 1)
        def _():
            pltpu.sync_copy(z2.at[pl.ds(0, rpt_g)],
                            gcnt_sh.at[pl.ds(s * rpt_g, rpt_g)])
            plsc.subcore_barrier()
            run(dst2, nck_e * NTILE, cnt_sh, cnt_b, nck_e, 6, rpt_n)
            run(pdst2, 0, gcnt_sh, gcnt_out, nck_p, 2, rpt_g)

    f32 = jnp.float32
    return pl.kernel(
        body,
        out_type=(jax.ShapeDtypeStruct((NPAD, 32), f32),
                  jax.ShapeDtypeStruct((NPAD, 32), f32),
                  jax.ShapeDtypeStruct((GPAD, 32), f32)),
        mesh=mesh,
        compiler_params=pltpu.CompilerParams(use_tc_tiling_on_sc=False),
        scratch_types=(
            pltpu.VMEM_SHARED((NPAD, 32), f32),
            pltpu.VMEM_SHARED((GPAD, 32), f32),
            pltpu.VMEM((8, CHUNK), jnp.int32),
            pltpu.VMEM((CHUNK, 32), f32),
            pltpu.SemaphoreType.DMA,
        ),
    )


def _embed_body(x0, x1, x2, bmat_p, lo_ref, hi_ref):
    i8 = lax.broadcasted_iota(jnp.int32, (PBLK, 8), 1)
    i20 = lax.broadcasted_iota(jnp.int32, (PBLK, 20), 1)
    groups = []
    for k in range(4):
        v0 = x0[0, k, :]
        v1 = x1[0, k, :]
        v2 = jnp.clip(x2[0, k, :], 0, MAX_POS - 1)
        groups.append(jnp.concatenate(
            [(v0[:, None] == i8).astype(jnp.float32),
             (v1[:, None] == i8).astype(jnp.float32),
             (v2[:, None] == i20).astype(jnp.float32),
             jnp.zeros((PBLK, 4), jnp.float32)], axis=1))
    ohp = jnp.concatenate(groups, axis=1)
    feat = jnp.dot(ohp, bmat_p[...], preferred_element_type=jnp.float32)
    lo_ref[...] = feat[:, :128]
    hi_ref[...] = feat[:, 128:]


def _sage_body(alo, ahi, flo, fhi, ca, cb, wl, wr, b, olo_ref, ohi_ref):
    acc = jnp.concatenate([alo[...], ahi[...]], axis=1)
    f = jnp.concatenate([flo[...], fhi[...]], axis=1)
    cnt = ca[...] + cb[...]
    inv = 1.0 / jnp.maximum(jnp.concatenate([cnt, cnt], axis=1), 1.0)
    h = (jnp.dot(acc, wl[...], preferred_element_type=jnp.float32) * inv
         + jnp.dot(f, wr[...], preferred_element_type=jnp.float32) + b[...])
    h = jnp.maximum(h, 0.0)
    olo_ref[...] = h[:, :128]
    ohi_ref[...] = h[:, 128:]


def _head_body(alo, ahi, g32, wc, bc, out_ref):
    acc = jnp.concatenate([alo[...], ahi[...]], axis=1)
    inv = 1.0 / jnp.maximum(g32[...][:, 0:1], 1.0)
    hg = acc * inv
    o = jnp.dot(hg, wc[...], preferred_element_type=jnp.float32) + bc[...]
    out_ref[...] = o[:G, :]


def _row_spec():
    return pl.BlockSpec((PBLK, 128), lambda i: (i, 0))


def _full_spec(shape):
    nd = len(shape)
    return pl.BlockSpec(shape, lambda i: (0,) * nd)


def kernel(x, edge_index, batch, shape_emb, color_emb, pos_emb,
           W1l, W1r, b1, W2l, W2r, b2, Wc, bc):
    f32 = jnp.float32
    i32 = jnp.int32

    xp = jnp.concatenate([x.astype(i32), jnp.zeros((NPAD - N, 3), i32)], axis=0)

    def xq(col):
        v = xp[:, col].reshape(NBLK, PBLK, 4).transpose(0, 2, 1)
        return jnp.concatenate([v, jnp.zeros((NBLK, 4, PBLK), i32)], axis=1)

    x0, x1, x2 = xq(0), xq(1), xq(2)

    bmat = jnp.zeros((40, 64), f32)
    bmat = bmat.at[0:8, 0:16].set(shape_emb)
    bmat = bmat.at[8:16, 16:32].set(color_emb)
    bmat = bmat.at[16:36, 32:48].set(pos_emb)
    bmat_p = _pack_bmat(bmat)

    w1l = _pack_w(jnp.zeros((64, 64), f32).at[:48].set(W1l))
    w1r = _pack_w(jnp.zeros((64, 64), f32).at[:48].set(W1r))
    w2l = _pack_w(W2l)
    w2r = _pack_w(W2r)
    b1p = _pack_b(b1)
    b2p = _pack_b(b2)
    bcr = bc.reshape(1, NUM_CLASS)

    src = edge_index[0].astype(i32)
    dst = edge_index[1].astype(i32)
    pad_cycle = jnp.arange(EPAD - E, dtype=i32) % 128
    src2 = jnp.concatenate([src, pad_cycle]).reshape(EPAD // CHUNK, CHUNK)
    dst2 = jnp.concatenate([dst, DUMMY_NODE + pad_cycle]).reshape(EPAD // CHUNK, CHUNK)

    ppad_cycle = jnp.arange(PEPAD - N, dtype=i32) % 128
    psrc = jnp.concatenate([jnp.arange(N, dtype=i32),
                            ppad_cycle]).reshape(PEPAD // CHUNK, CHUNK)
    pdst = jnp.concatenate([batch.astype(i32),
                            DUMMY_GRAPH + ppad_cycle]).reshape(PEPAD // CHUNK, CHUNK)

    ones32 = jnp.ones((CHUNK, 32), f32)
    zn2 = jnp.zeros((NPAD // NTILE, 32), f32)

    hist = _sc_hist_pass()
    cnt32a, cnt32b, gcnt32 = hist(dst2, pdst, zn2, ones32)
    c32a = jnp.reshape(cnt32a, (NPAD // 4, 128))
    c32b = jnp.reshape(cnt32b, (NPAD // 4, 128))

    embed = pl.pallas_call(
        _embed_body,
        grid=(NBLK,),
        in_specs=[pl.BlockSpec((1, 8, PBLK), lambda i: (i, 0, 0))] * 3
        + [_full_spec((160, 256))],
        out_specs=[_row_spec(), _row_spec()],
        out_shape=[jax.ShapeDtypeStruct((NPAD // 4, 128), f32)] * 2,
    )
    f0lo, f0hi = embed(x0, x1, x2, bmat_p)

    edge_pass = _sc_scatter_pass(EPAD, NPAD, 6)
    a1lo, a1hi = edge_pass(src2, dst2,
                           jnp.reshape(f0lo, (NPAD, 32)),
                           jnp.reshape(f0hi, (NPAD, 32)), zn2)

    sage = pl.pallas_call(
        _sage_body,
        grid=(NBLK,),
        in_specs=[_row_spec()] * 6
        + [_full_spec((256, 256)), _full_spec((256, 256)), _full_spec((1, 256))],
        out_specs=[_row_spec(), _row_spec()],
        out_shape=[jax.ShapeDtypeStruct((NPAD // 4, 128), f32)] * 2,
    )
    p1lo = jnp.reshape(a1lo, (NPAD // 4, 128))
    p1hi = jnp.reshape(a1hi, (NPAD // 4, 128))
    h1lo, h1hi = sage(p1lo, p1hi, f0lo, f0hi, c32a, c32b, w1l, w1r, b1p)

    a2lo, a2hi = edge_pass(src2, dst2,
                           jnp.reshape(h1lo, (NPAD, 32)),
                           jnp.reshape(h1hi, (NPAD, 32)), zn2)

    p2lo = jnp.reshape(a2lo, (NPAD // 4, 128))
    p2hi = jnp.reshape(a2hi, (NPAD // 4, 128))
    h2lo, h2hi = sage(p2lo, p2hi, h1lo, h1hi, c32a, c32b, w2l, w2r, b2p)

    pool_pass = _sc_scatter_pass(PEPAD, GPAD, 2)
    aplo, aphi = pool_pass(psrc, pdst,
                           jnp.reshape(h2lo, (NPAD, 32)),
                           jnp.reshape(h2hi, (NPAD, 32)), zn2)

    head = pl.pallas_call(
        _head_body,
        grid=(1,),
        in_specs=[_full_spec((GPAD, 32)), _full_spec((GPAD, 32)),
                  _full_spec((GPAD, 32)),
                  _full_spec((64, NUM_CLASS)), _full_spec((1, NUM_CLASS))],
        out_specs=_full_spec((G, NUM_CLASS)),
        out_shape=jax.ShapeDtypeStruct((G, NUM_CLASS), f32),
    )
    return head(aplo, aphi, gcnt32, Wc, bcr)
